# baseline (speedup 1.0000x reference)
import jax
import jax.numpy as jnp
from jax.experimental import pallas as pl
from jax.experimental.pallas import tpu as pltpu

_D = 64
_N = 8192
_HALF = 4096
_MBLK = 256


def _vq_kernel(x_ref, eb_ref, ethi_ref, etlo_ref, s1_ref, s2_ref,
               qst_ref, idx_ref, diff_ref):
    i = pl.program_id(0)
    nsteps = pl.num_programs(0)
    x = x_ref[...]
    xb = (x * 2.0).astype(jnp.bfloat16)
    mm = jnp.dot(xb, eb_ref[...], preferred_element_type=jnp.float32)
    d = (s1_ref[...] - mm) + s2_ref[...]

    d0 = d[:, :_HALF]
    d1 = d[:, _HALF:]
    md0 = jnp.min(d0, axis=1, keepdims=True)
    md1 = jnp.min(d1, axis=1, keepdims=True)
    ih = jax.lax.broadcasted_iota(jnp.int32, (d0.shape[0], _HALF), 1)
    i0 = jnp.min(jnp.where(d0 == md0, ih, jnp.int32(_N)), axis=1,
                 keepdims=True)
    i1 = jnp.min(jnp.where(d1 == md1, ih, jnp.int32(_N)), axis=1,
                 keepdims=True) + _HALF
    kb0 = md0.astype(jnp.bfloat16).astype(jnp.float32)
    keep0 = kb0 <= md1
    idx = jnp.where(keep0, i0, i1)
    idx_ref[...] = idx

    ii = jax.lax.broadcasted_iota(jnp.int32, d.shape, 1)
    onehot = jnp.where(ii == idx, jnp.float32(1.0),
                       jnp.float32(0.0)).astype(jnp.bfloat16)
    q = (jnp.dot(onehot, ethi_ref[...], preferred_element_type=jnp.float32)
         + jnp.dot(onehot, etlo_ref[...], preferred_element_type=jnp.float32))
    qst_ref[...] = x + (q - x)

    @pl.when(i == 0)
    def _init():
        diff_ref[...] = jnp.zeros((1, 1), jnp.float32)

    diff_ref[...] += jnp.sum((q - x) ** 2, keepdims=True)

    @pl.when(i == nsteps - 1)
    def _fin():
        diff_ref[...] *= jnp.float32(1.0 / (16384.0 * _D))


def kernel(input, embed):
    flatten = input.reshape(-1, _D)
    s1 = jnp.sum(flatten ** 2, axis=1, keepdims=True)
    s2 = jnp.sum(embed ** 2, axis=0, keepdims=True)
    eb = embed.astype(jnp.bfloat16)
    et = embed.T
    ethi = et.astype(jnp.bfloat16)
    etlo = (et - ethi.astype(jnp.float32)).astype(jnp.bfloat16)
    ntok = flatten.shape[0]
    grid = ntok // _MBLK
    qst, idx, diff = pl.pallas_call(
        _vq_kernel,
        grid=(grid,),
        in_specs=[
            pl.BlockSpec((_MBLK, _D), lambda i: (i, 0)),
            pl.BlockSpec((_D, _N), lambda i: (0, 0)),
            pl.BlockSpec((_N, _D), lambda i: (0, 0)),
            pl.BlockSpec((_N, _D), lambda i: (0, 0)),
            pl.BlockSpec((_MBLK, 1), lambda i: (i, 0)),
            pl.BlockSpec((1, _N), lambda i: (0, 0)),
        ],
        out_specs=[
            pl.BlockSpec((_MBLK, _D), lambda i: (i, 0)),
            pl.BlockSpec((_MBLK, 1), lambda i: (i, 0)),
            pl.BlockSpec((1, 1), lambda i: (0, 0)),
        ],
        out_shape=[
            jax.ShapeDtypeStruct((ntok, _D), jnp.float32),
            jax.ShapeDtypeStruct((ntok, 1), jnp.int32),
            jax.ShapeDtypeStruct((1, 1), jnp.float32),
        ],
        compiler_params=pltpu.CompilerParams(
            dimension_semantics=("arbitrary",),
        ),
    )(flatten, eb, ethi, etlo, s1, s2)
    quantize_st = qst.reshape(input.shape)
    embed_ind = idx.reshape(input.shape[:-1])
    return (quantize_st, diff.reshape(()), jnp.zeros((1,), jnp.float32),
            embed_ind)


# speedup vs baseline: 1.4225x; 1.4225x over previous
import jax
import jax.numpy as jnp
from jax.experimental import pallas as pl
from jax.experimental.pallas import tpu as pltpu

_D = 64
_N = 8192
_HALF = 4096
_MBLK = 256


def _argmin_kernel(x_ref, eb_ref, s1_ref, s2_ref, idx_ref):
    x = x_ref[...]
    xb = (x * 2.0).astype(jnp.bfloat16)
    mm = jnp.dot(xb, eb_ref[...], preferred_element_type=jnp.float32)
    d = (s1_ref[...] - mm) + s2_ref[...]

    d0 = d[:, :_HALF]
    d1 = d[:, _HALF:]
    md0 = jnp.min(d0, axis=1, keepdims=True)
    md1 = jnp.min(d1, axis=1, keepdims=True)
    ih = jax.lax.broadcasted_iota(jnp.int32, (d0.shape[0], _HALF), 1)
    i0 = jnp.min(jnp.where(d0 == md0, ih, jnp.int32(_N)), axis=1,
                 keepdims=True)
    i1 = jnp.min(jnp.where(d1 == md1, ih, jnp.int32(_N)), axis=1,
                 keepdims=True) + _HALF
    kb0 = md0.astype(jnp.bfloat16).astype(jnp.float32)
    idx_ref[...] = jnp.where(kb0 <= md1, i0, i1)


def _lookup_kernel(sidx_ref, x_ref, et_ref, qst_ref, diff_ref, qscr_ref):
    i = pl.program_id(0)
    nsteps = pl.num_programs(0)
    base = i * _MBLK

    def body(t, _):
        s = sidx_ref[base + t]
        qscr_ref[pl.ds(t, 1), :] = et_ref[pl.ds(s, 1), :]
        return _

    jax.lax.fori_loop(0, _MBLK, body, 0, unroll=8)
    x = x_ref[...]
    q = qscr_ref[...]
    qst_ref[...] = x + (q - x)

    @pl.when(i == 0)
    def _init():
        diff_ref[...] = jnp.zeros((1, 1), jnp.float32)

    diff_ref[...] += jnp.sum((q - x) ** 2, keepdims=True)

    @pl.when(i == nsteps - 1)
    def _fin():
        diff_ref[...] *= jnp.float32(1.0 / (16384.0 * _D))


def kernel(input, embed):
    flatten = input.reshape(-1, _D)
    s1 = jnp.sum(flatten ** 2, axis=1, keepdims=True)
    s2 = jnp.sum(embed ** 2, axis=0, keepdims=True)
    eb = embed.astype(jnp.bfloat16)
    et = embed.T
    ntok = flatten.shape[0]
    grid = ntok // _MBLK
    idx = pl.pallas_call(
        _argmin_kernel,
        grid=(grid,),
        in_specs=[
            pl.BlockSpec((_MBLK, _D), lambda i: (i, 0)),
            pl.BlockSpec((_D, _N), lambda i: (0, 0)),
            pl.BlockSpec((_MBLK, 1), lambda i: (i, 0)),
            pl.BlockSpec((1, _N), lambda i: (0, 0)),
        ],
        out_specs=pl.BlockSpec((_MBLK, 1), lambda i: (i, 0)),
        out_shape=jax.ShapeDtypeStruct((ntok, 1), jnp.int32),
        compiler_params=pltpu.CompilerParams(
            dimension_semantics=("arbitrary",),
        ),
    )(flatten, eb, s1, s2)

    qst, diff = pl.pallas_call(
        _lookup_kernel,
        grid_spec=pltpu.PrefetchScalarGridSpec(
            num_scalar_prefetch=1,
            grid=(grid,),
            in_specs=[
                pl.BlockSpec((_MBLK, _D), lambda i, s: (i, 0)),
                pl.BlockSpec((_N, _D), lambda i, s: (0, 0)),
            ],
            out_specs=[
                pl.BlockSpec((_MBLK, _D), lambda i, s: (i, 0)),
                pl.BlockSpec((1, 1), lambda i, s: (0, 0)),
            ],
            scratch_shapes=[pltpu.VMEM((_MBLK, _D), jnp.float32)],
        ),
        out_shape=[
            jax.ShapeDtypeStruct((ntok, _D), jnp.float32),
            jax.ShapeDtypeStruct((1, 1), jnp.float32),
        ],
        compiler_params=pltpu.CompilerParams(
            dimension_semantics=("arbitrary",),
        ),
    )(idx.reshape(-1), flatten, et)

    quantize_st = qst.reshape(input.shape)
    embed_ind = idx.reshape(input.shape[:-1])
    return (quantize_st, diff.reshape(()), jnp.zeros((1,), jnp.float32),
            embed_ind)
